# Initial kernel scaffold; baseline (speedup 1.0000x reference)
import jax
import jax.numpy as jnp
from jax.experimental import pallas as pl


def kernel(dist_exp, atom_emb, edge_index, bilinear_w, bilinear_b):
    raise NotImplementedError("write your pallas kernel here")


# speedup vs baseline: 3.5637x; 3.5637x over previous
import functools

import jax
import jax.numpy as jnp
from jax import lax
from jax.experimental import pallas as pl
from jax.experimental.pallas import tpu as pltpu
from jax.experimental.pallas import tpu_sc as plsc

N = 10000
E = 320000
F = 128
H = 128
K = 128


NB = 400
FC = 16


def _feat_kernel(d_ref, a_ref, w_ref, o_ref):
    a = a_ref[...]
    d = d_ref[...]
    a_t = jnp.concatenate([a] * FC, axis=1)
    acc = jnp.zeros((NB, K), jnp.float32)
    for fc in range(0, F, FC):
        d_rep = jnp.concatenate(
            [lax.broadcast_in_dim(d[:, fc + i][:, None], (NB, H), (0, 1))
             for i in range(FC)], axis=1)
        pair = d_rep * a_t
        acc = acc + jnp.dot(pair, w_ref[fc * H:(fc + FC) * H, :],
                            preferred_element_type=jnp.float32)
    o_ref[...] = acc


def _feat_src(dist_exp, atom_emb, w2d):
    grid = (N // NB,)
    return pl.pallas_call(
        _feat_kernel,
        grid=grid,
        in_specs=[
            pl.BlockSpec((NB, F), lambda i: (i, 0)),
            pl.BlockSpec((NB, H), lambda i: (i, 0)),
            pl.BlockSpec((F * H, K), lambda i: (0, 0)),
        ],
        out_specs=pl.BlockSpec((NB, K), lambda i: (i, 0)),
        out_shape=jax.ShapeDtypeStruct((N, K), jnp.float32),
    )(dist_exp, atom_emb, w2d)


NC = 2
NS = 16
CH = 80
EPT = E // (NC * NS)
NCHUNK = EPT // CH
STRIPE = 624
TAILROWS = N - NS * STRIPE


def _agg_kernel(feat_hbm, src_hbm, dst_hbm, zero_hbm, out_hbm,
                sidx_v, didx_v, rows_v, acc_sh, sem):
    c = lax.axis_index("c")
    s = lax.axis_index("s")
    pltpu.sync_copy(zero_hbm.at[pl.ds(s * STRIPE, STRIPE)],
                    acc_sh.at[pl.ds(s * STRIPE, STRIPE)])

    @pl.when(s == NS - 1)
    def _init_tail():
        pltpu.sync_copy(zero_hbm.at[pl.ds(NS * STRIPE, TAILROWS)],
                        acc_sh.at[pl.ds(NS * STRIPE, TAILROWS)])

    plsc.subcore_barrier()

    base = c * (E // NC) + s * EPT

    def body(j, carry):
        off = base + j * CH
        pltpu.sync_copy(src_hbm.at[pl.ds(off, CH)], sidx_v)
        pltpu.sync_copy(dst_hbm.at[pl.ds(off, CH)], didx_v)
        pltpu.async_copy(feat_hbm.at[sidx_v], rows_v, sem).wait()
        pltpu.sync_copy(rows_v, acc_sh.at[didx_v], add=True)
        return carry

    lax.fori_loop(0, NCHUNK, body, 0)
    plsc.subcore_barrier()
    pltpu.sync_copy(acc_sh.at[pl.ds(s * STRIPE, STRIPE)],
                    out_hbm.at[pl.ds(c * N + s * STRIPE, STRIPE)])

    @pl.when(s == NS - 1)
    def _out_tail():
        pltpu.sync_copy(acc_sh.at[pl.ds(NS * STRIPE, TAILROWS)],
                        out_hbm.at[pl.ds(c * N + NS * STRIPE, TAILROWS)])


def _aggregate(feat, src, dst, zeros_init):
    mesh = plsc.VectorSubcoreMesh(core_axis_name="c", subcore_axis_name="s")
    kern = pl.kernel(
        _agg_kernel,
        out_type=jax.ShapeDtypeStruct((NC * N, K), jnp.float32),
        mesh=mesh,
        scratch_types=[
            pltpu.VMEM((CH,), jnp.int32),
            pltpu.VMEM((CH,), jnp.int32),
            pltpu.VMEM((CH, K), jnp.float32),
            pltpu.VMEM_SHARED((N, K), jnp.float32),
            pltpu.SemaphoreType.DMA,
        ],
    )
    return kern(feat, src, dst, zeros_init)


def _tail_kernel(p0_ref, p1_ref, f_ref, b_ref, o_ref):
    h = p0_ref[...] + p1_ref[...] - f_ref[...]
    sp = jnp.maximum(h, 0.0) + jnp.log1p(jnp.exp(-jnp.abs(h)))
    o_ref[...] = sp + b_ref[...]


def _tail(partials, feat, bias):
    grid = (N // NB,)
    nb0 = N // NB
    return pl.pallas_call(
        _tail_kernel,
        grid=grid,
        in_specs=[
            pl.BlockSpec((NB, K), lambda i: (i, 0)),
            pl.BlockSpec((NB, K), lambda i, nb0=nb0: (i + nb0, 0)),
            pl.BlockSpec((NB, K), lambda i: (i, 0)),
            pl.BlockSpec((1, K), lambda i: (0, 0)),
        ],
        out_specs=pl.BlockSpec((NB, K), lambda i: (i, 0)),
        out_shape=jax.ShapeDtypeStruct((N, K), jnp.float32),
    )(partials, partials, feat, bias)


def kernel(dist_exp, atom_emb, edge_index, bilinear_w, bilinear_b):
    w2d = bilinear_w.reshape(F * H, K)
    feat = _feat_src(dist_exp, atom_emb, w2d)
    src = edge_index[0].astype(jnp.int32)
    dst = edge_index[1].astype(jnp.int32)
    zeros_init = jnp.zeros((N, K), jnp.float32)
    partials = _aggregate(feat, src, dst, zeros_init)
    bias = bilinear_b.reshape(1, K).astype(jnp.float32)
    return _tail(partials, feat, bias)
